# baseline (speedup 1.0000x reference)
import functools

import jax
import jax.numpy as jnp
from jax import lax
from jax.experimental import pallas as pl
from jax.experimental.pallas import tpu as pltpu
from jax.experimental.pallas import tpu_sc as plsc

NC = 2
NS = 16
NW = NC * NS
SUB = 80
KS = 8

N = 10000
E = 320000
D = 128
IDX_PER_W = 128
IDX_ROWS = NW * IDX_PER_W
E_PAD = IDX_ROWS * SUB
TRASH = 112
ACC_ROWS = N + TRASH
RPS = ACC_ROWS // NS

_mesh = plsc.VectorSubcoreMesh(
    core_axis_name="c", subcore_axis_name="s", num_cores=NC, num_subcores=NS)


@functools.partial(
    pl.kernel,
    out_type=jax.ShapeDtypeStruct((NC, ACC_ROWS, D), jnp.float32),
    mesh=_mesh,
    scratch_types=[
        pltpu.VMEM((KS, SUB), jnp.int32),
        pltpu.VMEM((KS, SUB), jnp.int32),
        pltpu.VMEM((SUB, D), jnp.float32),
        pltpu.VMEM((SUB, D), jnp.float32),
        pltpu.VMEM((SUB, D), jnp.float32),
        pltpu.VMEM((SUB, D), jnp.float32),
        pltpu.SemaphoreType.DMA,
        pltpu.SemaphoreType.DMA,
        pltpu.SemaphoreType.DMA,
        pltpu.SemaphoreType.DMA,
        pltpu.SemaphoreType.DMA,
        pltpu.SemaphoreType.DMA,
        pltpu.VMEM_SHARED((ACC_ROWS, D), jnp.float32),
    ],
)
def _sc_msg_segsum(ea_hbm, h_hbm, src_hbm, dst_hbm, zeros_hbm, out_hbm,
                   sidx_v, didx_v, be0, be1, bh0, bh1,
                   seme0, seme1, semh0, semh1, sems0, sems1, acc):
    c = lax.axis_index("c")
    s = lax.axis_index("s")
    wid = s * NC + c
    pltpu.sync_copy(zeros_hbm.at[pl.ds(s * RPS, RPS)],
                    acc.at[pl.ds(s * RPS, RPS)])
    plsc.subcore_barrier()
    base = wid * IDX_PER_W
    bes = (be0, be1)
    bhs = (bh0, bh1)
    semes = (seme0, seme1)
    semhs = (semh0, semh1)
    semss = (sems0, sems1)

    @pl.loop(0, IDX_PER_W, step=KS)
    def _(i):
        pltpu.sync_copy(src_hbm.at[pl.ds(base + i, KS)], sidx_v)
        pltpu.sync_copy(dst_hbm.at[pl.ds(base + i, KS)], didx_v)

        def start_in(j):
            b = j % 2
            ce = pltpu.async_copy(
                ea_hbm.at[pl.ds((base + i + j) * SUB, SUB)], bes[b], semes[b])
            ch = pltpu.async_copy(h_hbm.at[sidx_v.at[j]], bhs[b], semhs[b])
            return ce, ch

        ss = [None, None]
        cur = start_in(0)
        for j in range(KS):
            b = j % 2
            nxt = None
            if j + 1 < KS:
                if ss[1 - b] is not None:
                    ss[1 - b].wait()
                nxt = start_in(j + 1)
            cur[0].wait()
            cur[1].wait()

            @plsc.parallel_loop(0, SUB, unroll=4)
            def _(r):
                for cc in range(D // 16):
                    plsc.addupdate(bes[b].at[r, pl.ds(cc * 16, 16)],
                                   bhs[b][r, pl.ds(cc * 16, 16)])

            ss[b] = pltpu.async_copy(bes[b], acc.at[didx_v.at[j]], semss[b],
                                     add=True)
            cur = nxt
        ss[0].wait()
        ss[1].wait()

    plsc.subcore_barrier()
    pltpu.sync_copy(acc.at[pl.ds(s * RPS, RPS)],
                    out_hbm.at[c, pl.ds(s * RPS, RPS)])


def _mlp_block(xb, w1_ref, b1_ref, w2_ref, b2_ref, g_ref, beta_ref):
    w1 = w1_ref[...].astype(jnp.bfloat16)
    w2 = w2_ref[...].astype(jnp.bfloat16)
    h = jnp.dot(xb.astype(jnp.bfloat16), w1, preferred_element_type=jnp.float32)
    h = jnp.maximum(h + b1_ref[...], 0.0)
    h = jnp.dot(h.astype(jnp.bfloat16), w2, preferred_element_type=jnp.float32)
    h = h + b2_ref[...]
    mu = jnp.mean(h, axis=-1, keepdims=True)
    var = jnp.mean((h - mu) ** 2, axis=-1, keepdims=True)
    return (h - mu) * lax.rsqrt(var + 1e-5) * g_ref[...] + beta_ref[...]


_BE = 2048


def _edge_mlp_body(x_ref, w1_ref, b1_ref, w2_ref, b2_ref, g_ref, beta_ref, o_ref):
    o_ref[...] = _mlp_block(x_ref[...], w1_ref, b1_ref, w2_ref, b2_ref,
                            g_ref, beta_ref)


def _edge_mlp(xa, w1, b1, w2, b2, g, beta):
    full = pl.BlockSpec((D, D), lambda i: (0, 0))
    vec = pl.BlockSpec((1, D), lambda i: (0, 0))
    return pl.pallas_call(
        _edge_mlp_body,
        grid=(E_PAD // _BE,),
        in_specs=[pl.BlockSpec((_BE, D), lambda i: (i, 0)),
                  full, vec, full, vec, vec, vec],
        out_specs=pl.BlockSpec((_BE, D), lambda i: (i, 0)),
        out_shape=jax.ShapeDtypeStruct((E_PAD, D), jnp.float32),
    )(xa, w1, b1.reshape(1, D), w2, b2.reshape(1, D),
      g.reshape(1, D), beta.reshape(1, D))


_BN = 1000


def _node_body(h_ref, g0_ref, g1_ref, w1_ref, b1_ref, w2_ref, b2_ref,
               g_ref, beta_ref, o_ref):
    aggr = g0_ref[...] + g1_ref[...]
    o_ref[...] = h_ref[...] + _mlp_block(aggr, w1_ref, b1_ref, w2_ref, b2_ref,
                                         g_ref, beta_ref)


def _node_update(h, gp, w1, b1, w2, b2, g, beta):
    full = pl.BlockSpec((D, D), lambda i: (0, 0))
    vec = pl.BlockSpec((1, D), lambda i: (0, 0))
    blk = pl.BlockSpec((_BN, D), lambda i: (i, 0))
    return pl.pallas_call(
        _node_body,
        grid=(N // _BN,),
        in_specs=[blk, blk, blk, full, vec, full, vec, vec, vec],
        out_specs=blk,
        out_shape=jax.ShapeDtypeStruct((N, D), jnp.float32),
    )(h, gp[0], gp[1], w1, b1.reshape(1, D), w2, b2.reshape(1, D),
      g.reshape(1, D), beta.reshape(1, D))


def kernel(x, edge_index, edge_attr,
           node_w1, node_b1, node_w2, node_b2, node_g, node_beta,
           edge_w1, edge_b1, edge_w2, edge_b2, edge_g, edge_beta):
    L = node_w1.shape[0]
    pad = E_PAD - E
    ar = jnp.arange(pad, dtype=jnp.int32)
    src2d = jnp.concatenate([edge_index[0], ar % N]).reshape(IDX_ROWS, SUB)
    dst2d = jnp.concatenate([edge_index[1], N + (ar % TRASH)]).reshape(IDX_ROWS, SUB)
    ea_pad = jnp.concatenate([edge_attr, jnp.zeros((pad, D), jnp.float32)])
    zeros1 = jnp.zeros((ACC_ROWS, D), jnp.float32)

    eas = [_edge_mlp(ea_pad, edge_w1[l], edge_b1[l], edge_w2[l], edge_b2[l],
                     edge_g[l], edge_beta[l]) for l in range(L)]

    h = x
    for l in range(L):
        gp = _sc_msg_segsum(eas[l], h, src2d, dst2d, zeros1)
        h = _node_update(h, gp, node_w1[l], node_b1[l], node_w2[l], node_b2[l],
                         node_g[l], node_beta[l])
    return h


# speedup vs baseline: 1.0086x; 1.0086x over previous
import functools

import jax
import jax.numpy as jnp
from jax import lax
from jax.experimental import pallas as pl
from jax.experimental.pallas import tpu as pltpu
from jax.experimental.pallas import tpu_sc as plsc

NC = 2
NS = 16
NW = NC * NS
SUB = 80
KS = 8

N = 10000
E = 320000
D = 128
IDX_PER_W = 128
IDX_ROWS = NW * IDX_PER_W
E_PAD = IDX_ROWS * SUB
TRASH = 112
ACC_ROWS = N + TRASH
RPS = ACC_ROWS // NS

_mesh = plsc.VectorSubcoreMesh(
    core_axis_name="c", subcore_axis_name="s", num_cores=NC, num_subcores=NS)


@functools.partial(
    pl.kernel,
    out_type=jax.ShapeDtypeStruct((NC, ACC_ROWS, D), jnp.float32),
    mesh=_mesh,
    scratch_types=[
        pltpu.VMEM((KS, SUB), jnp.int32),
        pltpu.VMEM((KS, SUB), jnp.int32),
        pltpu.VMEM((SUB, D), jnp.float32),
        pltpu.VMEM((SUB, D), jnp.float32),
        pltpu.VMEM((SUB, D), jnp.float32),
        pltpu.VMEM((SUB, D), jnp.float32),
        pltpu.SemaphoreType.DMA,
        pltpu.SemaphoreType.DMA,
        pltpu.SemaphoreType.DMA,
        pltpu.SemaphoreType.DMA,
        pltpu.SemaphoreType.DMA,
        pltpu.SemaphoreType.DMA,
        pltpu.VMEM_SHARED((ACC_ROWS, D), jnp.float32),
    ],
)
def _sc_msg_segsum(ea_hbm, h_hbm, src_hbm, dst_hbm, zeros_hbm, out_hbm,
                   sidx_v, didx_v, be0, be1, bh0, bh1,
                   seme0, seme1, semh0, semh1, sems0, sems1, acc):
    c = lax.axis_index("c")
    s = lax.axis_index("s")
    wid = s * NC + c
    pltpu.sync_copy(zeros_hbm.at[pl.ds(s * RPS, RPS)],
                    acc.at[pl.ds(s * RPS, RPS)])
    plsc.subcore_barrier()
    base = wid * IDX_PER_W
    bes = (be0, be1)
    bhs = (bh0, bh1)
    semes = (seme0, seme1)
    semhs = (semh0, semh1)
    semss = (sems0, sems1)

    @pl.loop(0, IDX_PER_W, step=KS)
    def _(i):
        pltpu.sync_copy(src_hbm.at[pl.ds(base + i, KS)], sidx_v)
        pltpu.sync_copy(dst_hbm.at[pl.ds(base + i, KS)], didx_v)

        def start_in(j):
            b = j % 2
            ce = pltpu.async_copy(
                ea_hbm.at[pl.ds((base + i + j) * SUB, SUB)], bes[b], semes[b])
            ch = pltpu.async_copy(h_hbm.at[sidx_v.at[j]], bhs[b], semhs[b])
            return ce, ch

        ss = [None, None]
        cur = start_in(0)
        for j in range(KS):
            b = j % 2
            nxt = None
            if j + 1 < KS:
                if ss[1 - b] is not None:
                    ss[1 - b].wait()
                nxt = start_in(j + 1)
            cur[0].wait()
            cur[1].wait()

            @plsc.parallel_loop(0, SUB, unroll=4)
            def _(r):
                for cc in range(D // 16):
                    plsc.addupdate(bes[b].at[r, pl.ds(cc * 16, 16)],
                                   bhs[b][r, pl.ds(cc * 16, 16)])

            ss[b] = pltpu.async_copy(bes[b], acc.at[didx_v.at[j]], semss[b],
                                     add=True)
            cur = nxt
        ss[0].wait()
        ss[1].wait()

    plsc.subcore_barrier()
    pltpu.sync_copy(acc.at[pl.ds(s * RPS, RPS)],
                    out_hbm.at[c, pl.ds(s * RPS, RPS)])


def _mlp_block(xb, w1_ref, b1_ref, w2_ref, b2_ref, g_ref, beta_ref):
    w1 = w1_ref[...].astype(jnp.bfloat16)
    w2 = w2_ref[...].astype(jnp.bfloat16)
    h = jnp.dot(xb.astype(jnp.bfloat16), w1, preferred_element_type=jnp.float32)
    h = jnp.maximum(h + b1_ref[...], 0.0)
    h = jnp.dot(h.astype(jnp.bfloat16), w2, preferred_element_type=jnp.float32)
    h = h + b2_ref[...]
    mu = jnp.mean(h, axis=-1, keepdims=True)
    var = jnp.mean((h - mu) ** 2, axis=-1, keepdims=True)
    return (h - mu) * lax.rsqrt(var + 1e-5) * g_ref[...] + beta_ref[...]


_BE = 2048


def _edge_mlp_body(x_ref, w1_ref, b1_ref, w2_ref, b2_ref, g_ref, beta_ref, o_ref):
    o_ref[...] = _mlp_block(x_ref[...], w1_ref, b1_ref, w2_ref, b2_ref,
                            g_ref, beta_ref)


def _edge_mlp(xa, w1, b1, w2, b2, g, beta):
    full = pl.BlockSpec((D, D), lambda i: (0, 0))
    vec = pl.BlockSpec((1, D), lambda i: (0, 0))
    return pl.pallas_call(
        _edge_mlp_body,
        grid=(E_PAD // _BE,),
        in_specs=[pl.BlockSpec((_BE, D), lambda i: (i, 0)),
                  full, vec, full, vec, vec, vec],
        out_specs=pl.BlockSpec((_BE, D), lambda i: (i, 0)),
        out_shape=jax.ShapeDtypeStruct((E_PAD, D), jnp.float32),
    )(xa, w1, b1.reshape(1, D), w2, b2.reshape(1, D),
      g.reshape(1, D), beta.reshape(1, D))


_BN = 2000


def _node_body(h_ref, g0_ref, g1_ref, w1_ref, b1_ref, w2_ref, b2_ref,
               g_ref, beta_ref, o_ref):
    aggr = g0_ref[...] + g1_ref[...]
    o_ref[...] = h_ref[...] + _mlp_block(aggr, w1_ref, b1_ref, w2_ref, b2_ref,
                                         g_ref, beta_ref)


def _node_update(h, gp, w1, b1, w2, b2, g, beta):
    full = pl.BlockSpec((D, D), lambda i: (0, 0))
    vec = pl.BlockSpec((1, D), lambda i: (0, 0))
    blk = pl.BlockSpec((_BN, D), lambda i: (i, 0))
    return pl.pallas_call(
        _node_body,
        grid=(N // _BN,),
        in_specs=[blk, blk, blk, full, vec, full, vec, vec, vec],
        out_specs=blk,
        out_shape=jax.ShapeDtypeStruct((N, D), jnp.float32),
    )(h, gp[0], gp[1], w1, b1.reshape(1, D), w2, b2.reshape(1, D),
      g.reshape(1, D), beta.reshape(1, D))


def kernel(x, edge_index, edge_attr,
           node_w1, node_b1, node_w2, node_b2, node_g, node_beta,
           edge_w1, edge_b1, edge_w2, edge_b2, edge_g, edge_beta):
    L = node_w1.shape[0]
    pad = E_PAD - E
    ar = jnp.arange(pad, dtype=jnp.int32)
    src2d = jnp.concatenate([edge_index[0], ar % N]).reshape(IDX_ROWS, SUB)
    dst2d = jnp.concatenate([edge_index[1], N + (ar % TRASH)]).reshape(IDX_ROWS, SUB)
    ea_pad = jnp.concatenate([edge_attr, jnp.zeros((pad, D), jnp.float32)])
    zeros1 = jnp.zeros((ACC_ROWS, D), jnp.float32)

    eas = [_edge_mlp(ea_pad, edge_w1[l], edge_b1[l], edge_w2[l], edge_b2[l],
                     edge_g[l], edge_beta[l]) for l in range(L)]

    h = x
    for l in range(L):
        gp = _sc_msg_segsum(eas[l], h, src2d, dst2d, zeros1)
        h = _node_update(h, gp, node_w1[l], node_b1[l], node_w2[l], node_b2[l],
                         node_g[l], node_beta[l])
    return h


# speedup vs baseline: 1.0778x; 1.0686x over previous
import functools

import jax
import jax.numpy as jnp
from jax import lax
from jax.experimental import pallas as pl
from jax.experimental.pallas import tpu as pltpu
from jax.experimental.pallas import tpu_sc as plsc

NC = 2
NS = 16
NW = NC * NS
SUB = 80
KS = 16

N = 10000
E = 320000
D = 128
IDX_PER_W = 128
IDX_ROWS = NW * IDX_PER_W
E_PAD = IDX_ROWS * SUB
TRASH = 112
ACC_ROWS = N + TRASH
RPS = ACC_ROWS // NS

_mesh = plsc.VectorSubcoreMesh(
    core_axis_name="c", subcore_axis_name="s", num_cores=NC, num_subcores=NS)


@functools.partial(
    pl.kernel,
    out_type=jax.ShapeDtypeStruct((NC, ACC_ROWS, D), jnp.float32),
    mesh=_mesh,
    scratch_types=[
        pltpu.VMEM((KS, SUB), jnp.int32),
        pltpu.VMEM((KS, SUB), jnp.int32),
        pltpu.VMEM((SUB, D), jnp.float32),
        pltpu.VMEM((SUB, D), jnp.float32),
        pltpu.VMEM((SUB, D), jnp.float32),
        pltpu.VMEM((SUB, D), jnp.float32),
        pltpu.SemaphoreType.DMA,
        pltpu.SemaphoreType.DMA,
        pltpu.SemaphoreType.DMA,
        pltpu.SemaphoreType.DMA,
        pltpu.SemaphoreType.DMA,
        pltpu.SemaphoreType.DMA,
        pltpu.VMEM_SHARED((ACC_ROWS, D), jnp.float32),
    ],
)
def _sc_msg_segsum(ea_hbm, h_hbm, src_hbm, dst_hbm, zeros_hbm, out_hbm,
                   sidx_v, didx_v, be0, be1, bh0, bh1,
                   seme0, seme1, semh0, semh1, sems0, sems1, acc):
    c = lax.axis_index("c")
    s = lax.axis_index("s")
    wid = s * NC + c
    pltpu.sync_copy(zeros_hbm.at[pl.ds(s * RPS, RPS)],
                    acc.at[pl.ds(s * RPS, RPS)])
    plsc.subcore_barrier()
    base = wid * IDX_PER_W
    bes = (be0, be1)
    bhs = (bh0, bh1)
    semes = (seme0, seme1)
    semhs = (semh0, semh1)
    semss = (sems0, sems1)

    @pl.loop(0, IDX_PER_W, step=KS)
    def _(i):
        pltpu.sync_copy(src_hbm.at[pl.ds(base + i, KS)], sidx_v)
        pltpu.sync_copy(dst_hbm.at[pl.ds(base + i, KS)], didx_v)

        def start_in(j):
            b = j % 2
            ce = pltpu.async_copy(
                ea_hbm.at[pl.ds((base + i + j) * SUB, SUB)], bes[b], semes[b])
            ch = pltpu.async_copy(h_hbm.at[sidx_v.at[j]], bhs[b], semhs[b])
            return ce, ch

        ss = [None, None]
        cur = start_in(0)
        for j in range(KS):
            b = j % 2
            nxt = None
            if j + 1 < KS:
                if ss[1 - b] is not None:
                    ss[1 - b].wait()
                nxt = start_in(j + 1)
            cur[0].wait()
            cur[1].wait()

            @plsc.parallel_loop(0, SUB, unroll=4)
            def _(r):
                for cc in range(D // 16):
                    plsc.addupdate(bes[b].at[r, pl.ds(cc * 16, 16)],
                                   bhs[b][r, pl.ds(cc * 16, 16)])

            ss[b] = pltpu.async_copy(bes[b], acc.at[didx_v.at[j]], semss[b],
                                     add=True)
            cur = nxt
        ss[0].wait()
        ss[1].wait()

    plsc.subcore_barrier()
    pltpu.sync_copy(acc.at[pl.ds(s * RPS, RPS)],
                    out_hbm.at[c, pl.ds(s * RPS, RPS)])


def _mlp_block(xb, w1_ref, b1_ref, w2_ref, b2_ref, g_ref, beta_ref):
    w1 = w1_ref[...].astype(jnp.bfloat16)
    w2 = w2_ref[...].astype(jnp.bfloat16)
    h = jnp.dot(xb.astype(jnp.bfloat16), w1, preferred_element_type=jnp.float32)
    h = jnp.maximum(h + b1_ref[...], 0.0)
    h = jnp.dot(h.astype(jnp.bfloat16), w2, preferred_element_type=jnp.float32)
    h = h + b2_ref[...]
    mu = jnp.mean(h, axis=-1, keepdims=True)
    var = jnp.mean((h - mu) ** 2, axis=-1, keepdims=True)
    return (h - mu) * lax.rsqrt(var + 1e-5) * g_ref[...] + beta_ref[...]


_BE = 2048


def _edge_mlp_body(x_ref, w1_ref, b1_ref, w2_ref, b2_ref, g_ref, beta_ref, o_ref):
    o_ref[...] = _mlp_block(x_ref[...], w1_ref, b1_ref, w2_ref, b2_ref,
                            g_ref, beta_ref)


def _edge_mlp(xa, w1, b1, w2, b2, g, beta):
    full = pl.BlockSpec((D, D), lambda i: (0, 0))
    vec = pl.BlockSpec((1, D), lambda i: (0, 0))
    return pl.pallas_call(
        _edge_mlp_body,
        grid=(E_PAD // _BE,),
        in_specs=[pl.BlockSpec((_BE, D), lambda i: (i, 0)),
                  full, vec, full, vec, vec, vec],
        out_specs=pl.BlockSpec((_BE, D), lambda i: (i, 0)),
        out_shape=jax.ShapeDtypeStruct((E_PAD, D), jnp.float32),
    )(xa, w1, b1.reshape(1, D), w2, b2.reshape(1, D),
      g.reshape(1, D), beta.reshape(1, D))


_BN = 2000


def _node_body(h_ref, g0_ref, g1_ref, w1_ref, b1_ref, w2_ref, b2_ref,
               g_ref, beta_ref, o_ref):
    aggr = g0_ref[...] + g1_ref[...]
    o_ref[...] = h_ref[...] + _mlp_block(aggr, w1_ref, b1_ref, w2_ref, b2_ref,
                                         g_ref, beta_ref)


def _node_update(h, gp, w1, b1, w2, b2, g, beta):
    full = pl.BlockSpec((D, D), lambda i: (0, 0))
    vec = pl.BlockSpec((1, D), lambda i: (0, 0))
    blk = pl.BlockSpec((_BN, D), lambda i: (i, 0))
    return pl.pallas_call(
        _node_body,
        grid=(N // _BN,),
        in_specs=[blk, blk, blk, full, vec, full, vec, vec, vec],
        out_specs=blk,
        out_shape=jax.ShapeDtypeStruct((N, D), jnp.float32),
    )(h, gp[0], gp[1], w1, b1.reshape(1, D), w2, b2.reshape(1, D),
      g.reshape(1, D), beta.reshape(1, D))


def kernel(x, edge_index, edge_attr,
           node_w1, node_b1, node_w2, node_b2, node_g, node_beta,
           edge_w1, edge_b1, edge_w2, edge_b2, edge_g, edge_beta):
    L = node_w1.shape[0]
    pad = E_PAD - E
    ar = jnp.arange(pad, dtype=jnp.int32)
    src2d = jnp.concatenate([edge_index[0], ar % N]).reshape(IDX_ROWS, SUB)
    dst2d = jnp.concatenate([edge_index[1], N + (ar % TRASH)]).reshape(IDX_ROWS, SUB)
    ea_pad = jnp.concatenate([edge_attr, jnp.zeros((pad, D), jnp.float32)])
    zeros1 = jnp.zeros((ACC_ROWS, D), jnp.float32)

    eas = [_edge_mlp(ea_pad, edge_w1[l], edge_b1[l], edge_w2[l], edge_b2[l],
                     edge_g[l], edge_beta[l]) for l in range(L)]

    h = x
    for l in range(L):
        gp = _sc_msg_segsum(eas[l], h, src2d, dst2d, zeros1)
        h = _node_update(h, gp, node_w1[l], node_b1[l], node_w2[l], node_b2[l],
                         node_g[l], node_beta[l])
    return h


# speedup vs baseline: 1.0939x; 1.0149x over previous
import functools

import jax
import jax.numpy as jnp
from jax import lax
from jax.experimental import pallas as pl
from jax.experimental.pallas import tpu as pltpu
from jax.experimental.pallas import tpu_sc as plsc

NC = 2
NS = 16
NW = NC * NS
SUB = 80
KS = 32

N = 10000
E = 320000
D = 128
IDX_PER_W = 128
IDX_ROWS = NW * IDX_PER_W
E_PAD = IDX_ROWS * SUB
TRASH = 112
ACC_ROWS = N + TRASH
RPS = ACC_ROWS // NS

_mesh = plsc.VectorSubcoreMesh(
    core_axis_name="c", subcore_axis_name="s", num_cores=NC, num_subcores=NS)


@functools.partial(
    pl.kernel,
    out_type=jax.ShapeDtypeStruct((NC, ACC_ROWS, D), jnp.float32),
    mesh=_mesh,
    scratch_types=[
        pltpu.VMEM((KS, SUB), jnp.int32),
        pltpu.VMEM((KS, SUB), jnp.int32),
        pltpu.VMEM((SUB, D), jnp.float32),
        pltpu.VMEM((SUB, D), jnp.float32),
        pltpu.VMEM((SUB, D), jnp.float32),
        pltpu.VMEM((SUB, D), jnp.float32),
        pltpu.SemaphoreType.DMA,
        pltpu.SemaphoreType.DMA,
        pltpu.SemaphoreType.DMA,
        pltpu.SemaphoreType.DMA,
        pltpu.SemaphoreType.DMA,
        pltpu.SemaphoreType.DMA,
        pltpu.VMEM_SHARED((ACC_ROWS, D), jnp.float32),
    ],
)
def _sc_msg_segsum(ea_hbm, h_hbm, src_hbm, dst_hbm, zeros_hbm, out_hbm,
                   sidx_v, didx_v, be0, be1, bh0, bh1,
                   seme0, seme1, semh0, semh1, sems0, sems1, acc):
    c = lax.axis_index("c")
    s = lax.axis_index("s")
    wid = s * NC + c
    pltpu.sync_copy(zeros_hbm.at[pl.ds(s * RPS, RPS)],
                    acc.at[pl.ds(s * RPS, RPS)])
    plsc.subcore_barrier()
    base = wid * IDX_PER_W
    bes = (be0, be1)
    bhs = (bh0, bh1)
    semes = (seme0, seme1)
    semhs = (semh0, semh1)
    semss = (sems0, sems1)

    @pl.loop(0, IDX_PER_W, step=KS)
    def _(i):
        pltpu.sync_copy(src_hbm.at[pl.ds(base + i, KS)], sidx_v)
        pltpu.sync_copy(dst_hbm.at[pl.ds(base + i, KS)], didx_v)

        def start_in(j):
            b = j % 2
            ce = pltpu.async_copy(
                ea_hbm.at[pl.ds((base + i + j) * SUB, SUB)], bes[b], semes[b])
            ch = pltpu.async_copy(h_hbm.at[sidx_v.at[j]], bhs[b], semhs[b])
            return ce, ch

        ss = [None, None]
        cur = start_in(0)
        for j in range(KS):
            b = j % 2
            nxt = None
            if j + 1 < KS:
                if ss[1 - b] is not None:
                    ss[1 - b].wait()
                nxt = start_in(j + 1)
            cur[0].wait()
            cur[1].wait()

            @plsc.parallel_loop(0, SUB, unroll=4)
            def _(r):
                for cc in range(D // 16):
                    plsc.addupdate(bes[b].at[r, pl.ds(cc * 16, 16)],
                                   bhs[b][r, pl.ds(cc * 16, 16)])

            ss[b] = pltpu.async_copy(bes[b], acc.at[didx_v.at[j]], semss[b],
                                     add=True)
            cur = nxt
        ss[0].wait()
        ss[1].wait()

    plsc.subcore_barrier()
    pltpu.sync_copy(acc.at[pl.ds(s * RPS, RPS)],
                    out_hbm.at[c, pl.ds(s * RPS, RPS)])


def _mlp_block(xb, w1_ref, b1_ref, w2_ref, b2_ref, g_ref, beta_ref):
    w1 = w1_ref[...].astype(jnp.bfloat16)
    w2 = w2_ref[...].astype(jnp.bfloat16)
    h = jnp.dot(xb.astype(jnp.bfloat16), w1, preferred_element_type=jnp.float32)
    h = jnp.maximum(h + b1_ref[...], 0.0)
    h = jnp.dot(h.astype(jnp.bfloat16), w2, preferred_element_type=jnp.float32)
    h = h + b2_ref[...]
    mu = jnp.mean(h, axis=-1, keepdims=True)
    var = jnp.mean((h - mu) ** 2, axis=-1, keepdims=True)
    return (h - mu) * lax.rsqrt(var + 1e-5) * g_ref[...] + beta_ref[...]


_BE = 2048


def _edge_mlp_body(x_ref, w1_ref, b1_ref, w2_ref, b2_ref, g_ref, beta_ref, o_ref):
    o_ref[...] = _mlp_block(x_ref[...], w1_ref, b1_ref, w2_ref, b2_ref,
                            g_ref, beta_ref)


def _edge_mlp(xa, w1, b1, w2, b2, g, beta):
    full = pl.BlockSpec((D, D), lambda i: (0, 0))
    vec = pl.BlockSpec((1, D), lambda i: (0, 0))
    return pl.pallas_call(
        _edge_mlp_body,
        grid=(E_PAD // _BE,),
        in_specs=[pl.BlockSpec((_BE, D), lambda i: (i, 0)),
                  full, vec, full, vec, vec, vec],
        out_specs=pl.BlockSpec((_BE, D), lambda i: (i, 0)),
        out_shape=jax.ShapeDtypeStruct((E_PAD, D), jnp.float32),
    )(xa, w1, b1.reshape(1, D), w2, b2.reshape(1, D),
      g.reshape(1, D), beta.reshape(1, D))


_BN = 2000


def _node_body(h_ref, g0_ref, g1_ref, w1_ref, b1_ref, w2_ref, b2_ref,
               g_ref, beta_ref, o_ref):
    aggr = g0_ref[...] + g1_ref[...]
    o_ref[...] = h_ref[...] + _mlp_block(aggr, w1_ref, b1_ref, w2_ref, b2_ref,
                                         g_ref, beta_ref)


def _node_update(h, gp, w1, b1, w2, b2, g, beta):
    full = pl.BlockSpec((D, D), lambda i: (0, 0))
    vec = pl.BlockSpec((1, D), lambda i: (0, 0))
    blk = pl.BlockSpec((_BN, D), lambda i: (i, 0))
    return pl.pallas_call(
        _node_body,
        grid=(N // _BN,),
        in_specs=[blk, blk, blk, full, vec, full, vec, vec, vec],
        out_specs=blk,
        out_shape=jax.ShapeDtypeStruct((N, D), jnp.float32),
    )(h, gp[0], gp[1], w1, b1.reshape(1, D), w2, b2.reshape(1, D),
      g.reshape(1, D), beta.reshape(1, D))


def kernel(x, edge_index, edge_attr,
           node_w1, node_b1, node_w2, node_b2, node_g, node_beta,
           edge_w1, edge_b1, edge_w2, edge_b2, edge_g, edge_beta):
    L = node_w1.shape[0]
    pad = E_PAD - E
    ar = jnp.arange(pad, dtype=jnp.int32)
    src2d = jnp.concatenate([edge_index[0], ar % N]).reshape(IDX_ROWS, SUB)
    dst2d = jnp.concatenate([edge_index[1], N + (ar % TRASH)]).reshape(IDX_ROWS, SUB)
    ea_pad = jnp.concatenate([edge_attr, jnp.zeros((pad, D), jnp.float32)])
    zeros1 = jnp.zeros((ACC_ROWS, D), jnp.float32)

    eas = [_edge_mlp(ea_pad, edge_w1[l], edge_b1[l], edge_w2[l], edge_b2[l],
                     edge_g[l], edge_beta[l]) for l in range(L)]

    h = x
    for l in range(L):
        gp = _sc_msg_segsum(eas[l], h, src2d, dst2d, zeros1)
        h = _node_update(h, gp, node_w1[l], node_b1[l], node_w2[l], node_b2[l],
                         node_g[l], node_beta[l])
    return h


# speedup vs baseline: 1.0956x; 1.0016x over previous
import functools

import jax
import jax.numpy as jnp
from jax import lax
from jax.experimental import pallas as pl
from jax.experimental.pallas import tpu as pltpu
from jax.experimental.pallas import tpu_sc as plsc

NC = 2
NS = 16
NW = NC * NS
SUB = 80
KS = 32

N = 10000
E = 320000
D = 128
IDX_PER_W = 128
IDX_ROWS = NW * IDX_PER_W
E_PAD = IDX_ROWS * SUB
TRASH = 112
ACC_ROWS = N + TRASH
RPS = ACC_ROWS // NS

_mesh = plsc.VectorSubcoreMesh(
    core_axis_name="c", subcore_axis_name="s", num_cores=NC, num_subcores=NS)


@functools.partial(
    pl.kernel,
    out_type=jax.ShapeDtypeStruct((NC, ACC_ROWS, D), jnp.float32),
    mesh=_mesh,
    scratch_types=[
        pltpu.VMEM((KS, SUB), jnp.int32),
        pltpu.VMEM((KS, SUB), jnp.int32),
        pltpu.VMEM((SUB, D), jnp.float32),
        pltpu.VMEM((SUB, D), jnp.float32),
        pltpu.VMEM((SUB, D), jnp.float32),
        pltpu.VMEM((SUB, D), jnp.float32),
        pltpu.SemaphoreType.DMA,
        pltpu.SemaphoreType.DMA,
        pltpu.SemaphoreType.DMA,
        pltpu.SemaphoreType.DMA,
        pltpu.SemaphoreType.DMA,
        pltpu.SemaphoreType.DMA,
        pltpu.VMEM_SHARED((ACC_ROWS, D), jnp.float32),
    ],
)
def _sc_msg_segsum(ea_hbm, h_hbm, src_hbm, dst_hbm, zeros_hbm, out_hbm,
                   sidx_v, didx_v, be0, be1, bh0, bh1,
                   seme0, seme1, semh0, semh1, sems0, sems1, acc):
    c = lax.axis_index("c")
    s = lax.axis_index("s")
    wid = s * NC + c
    pltpu.sync_copy(zeros_hbm.at[pl.ds(s * RPS, RPS)],
                    acc.at[pl.ds(s * RPS, RPS)])
    plsc.subcore_barrier()
    base = wid * IDX_PER_W
    bes = (be0, be1)
    bhs = (bh0, bh1)
    semes = (seme0, seme1)
    semhs = (semh0, semh1)
    semss = (sems0, sems1)

    @pl.loop(0, IDX_PER_W, step=KS)
    def _(i):
        ci0 = pltpu.async_copy(src_hbm.at[pl.ds(base + i, KS)], sidx_v, sems0)
        ci1 = pltpu.async_copy(dst_hbm.at[pl.ds(base + i, KS)], didx_v, sems1)
        ce0 = pltpu.async_copy(
            ea_hbm.at[pl.ds((base + i) * SUB, SUB)], bes[0], semes[0])
        ci0.wait()
        ci1.wait()

        def start_in(j):
            b = j % 2
            ce = pltpu.async_copy(
                ea_hbm.at[pl.ds((base + i + j) * SUB, SUB)], bes[b], semes[b])
            ch = pltpu.async_copy(h_hbm.at[sidx_v.at[j]], bhs[b], semhs[b])
            return ce, ch

        ss = [None, None]
        ch0 = pltpu.async_copy(h_hbm.at[sidx_v.at[0]], bhs[0], semhs[0])
        cur = (ce0, ch0)
        for j in range(KS):
            b = j % 2
            nxt = None
            if j + 1 < KS:
                if ss[1 - b] is not None:
                    ss[1 - b].wait()
                nxt = start_in(j + 1)
            cur[0].wait()
            cur[1].wait()

            @plsc.parallel_loop(0, SUB, unroll=4)
            def _(r):
                for cc in range(D // 16):
                    plsc.addupdate(bes[b].at[r, pl.ds(cc * 16, 16)],
                                   bhs[b][r, pl.ds(cc * 16, 16)])

            ss[b] = pltpu.async_copy(bes[b], acc.at[didx_v.at[j]], semss[b],
                                     add=True)
            cur = nxt
        ss[0].wait()
        ss[1].wait()

    plsc.subcore_barrier()
    pltpu.sync_copy(acc.at[pl.ds(s * RPS, RPS)],
                    out_hbm.at[c, pl.ds(s * RPS, RPS)])


def _mlp_block(xb, w1_ref, b1_ref, w2_ref, b2_ref, g_ref, beta_ref):
    w1 = w1_ref[...].astype(jnp.bfloat16)
    w2 = w2_ref[...].astype(jnp.bfloat16)
    h = jnp.dot(xb.astype(jnp.bfloat16), w1, preferred_element_type=jnp.float32)
    h = jnp.maximum(h + b1_ref[...], 0.0)
    h = jnp.dot(h.astype(jnp.bfloat16), w2, preferred_element_type=jnp.float32)
    h = h + b2_ref[...]
    mu = jnp.mean(h, axis=-1, keepdims=True)
    var = jnp.mean((h - mu) ** 2, axis=-1, keepdims=True)
    return (h - mu) * lax.rsqrt(var + 1e-5) * g_ref[...] + beta_ref[...]


_BE = 2048


def _edge_mlp_body(x_ref, w1_ref, b1_ref, w2_ref, b2_ref, g_ref, beta_ref, o_ref):
    o_ref[...] = _mlp_block(x_ref[...], w1_ref, b1_ref, w2_ref, b2_ref,
                            g_ref, beta_ref)


def _edge_mlp(xa, w1, b1, w2, b2, g, beta):
    full = pl.BlockSpec((D, D), lambda i: (0, 0))
    vec = pl.BlockSpec((1, D), lambda i: (0, 0))
    return pl.pallas_call(
        _edge_mlp_body,
        grid=(E_PAD // _BE,),
        in_specs=[pl.BlockSpec((_BE, D), lambda i: (i, 0)),
                  full, vec, full, vec, vec, vec],
        out_specs=pl.BlockSpec((_BE, D), lambda i: (i, 0)),
        out_shape=jax.ShapeDtypeStruct((E_PAD, D), jnp.float32),
    )(xa, w1, b1.reshape(1, D), w2, b2.reshape(1, D),
      g.reshape(1, D), beta.reshape(1, D))


_BN = 2000


def _node_body(h_ref, g0_ref, g1_ref, w1_ref, b1_ref, w2_ref, b2_ref,
               g_ref, beta_ref, o_ref):
    aggr = g0_ref[...] + g1_ref[...]
    o_ref[...] = h_ref[...] + _mlp_block(aggr, w1_ref, b1_ref, w2_ref, b2_ref,
                                         g_ref, beta_ref)


def _node_update(h, gp, w1, b1, w2, b2, g, beta):
    full = pl.BlockSpec((D, D), lambda i: (0, 0))
    vec = pl.BlockSpec((1, D), lambda i: (0, 0))
    blk = pl.BlockSpec((_BN, D), lambda i: (i, 0))
    return pl.pallas_call(
        _node_body,
        grid=(N // _BN,),
        in_specs=[blk, blk, blk, full, vec, full, vec, vec, vec],
        out_specs=blk,
        out_shape=jax.ShapeDtypeStruct((N, D), jnp.float32),
    )(h, gp[0], gp[1], w1, b1.reshape(1, D), w2, b2.reshape(1, D),
      g.reshape(1, D), beta.reshape(1, D))


def kernel(x, edge_index, edge_attr,
           node_w1, node_b1, node_w2, node_b2, node_g, node_beta,
           edge_w1, edge_b1, edge_w2, edge_b2, edge_g, edge_beta):
    L = node_w1.shape[0]
    pad = E_PAD - E
    ar = jnp.arange(pad, dtype=jnp.int32)
    src2d = jnp.concatenate([edge_index[0], ar % N]).reshape(IDX_ROWS, SUB)
    dst2d = jnp.concatenate([edge_index[1], N + (ar % TRASH)]).reshape(IDX_ROWS, SUB)
    ea_pad = jnp.concatenate([edge_attr, jnp.zeros((pad, D), jnp.float32)])
    zeros1 = jnp.zeros((ACC_ROWS, D), jnp.float32)

    eas = [_edge_mlp(ea_pad, edge_w1[l], edge_b1[l], edge_w2[l], edge_b2[l],
                     edge_g[l], edge_beta[l]) for l in range(L)]

    h = x
    for l in range(L):
        gp = _sc_msg_segsum(eas[l], h, src2d, dst2d, zeros1)
        h = _node_update(h, gp, node_w1[l], node_b1[l], node_w2[l], node_b2[l],
                         node_g[l], node_beta[l])
    return h
